# initial kernel scaffold (unmeasured)
import jax
import jax.numpy as jnp
from jax import lax
from jax.experimental import pallas as pl
from jax.experimental.pallas import tpu as pltpu

N_DEV = 4
SQ = 512
D = 1024
H = 8
DH = 128
SKV = 2048
SCALE = 0.08838834764831843


def kernel(x, Wq, Wo, K_ext, V_ext):
    my = lax.axis_index("i")
    xb = x[0]
    k = lax.dynamic_slice_in_dim(K_ext[0], my * H, H, axis=1)
    v = lax.dynamic_slice_in_dim(V_ext[0], my * H, H, axis=1)
    k = jnp.transpose(k, (1, 0, 2))
    v = jnp.transpose(v, (1, 0, 2))

    def body(x_ref, wq_ref, wo_ref, k_ref, v_ref, out_ref,
             xfull, partial, rsrecv, ag_ssem, ag_rsem, rs_ssem, rs_rsem):
        my_i = lax.axis_index("i")
        left = lax.rem(my_i - 1 + N_DEV, N_DEV)
        right = lax.rem(my_i + 1, N_DEV)

        barrier = pltpu.get_barrier_semaphore()
        for nbr in (left, right):
            pl.semaphore_signal(barrier, inc=1, device_id=(nbr,),
                                device_id_type=pl.DeviceIdType.MESH)
        pl.semaphore_wait(barrier, 2)

        xfull[0, :, :] = x_ref[:, :]

        for h in range(N_DEV - 1):
            rdma = pltpu.make_async_remote_copy(
                src_ref=xfull.at[h],
                dst_ref=xfull.at[h + 1],
                send_sem=ag_ssem.at[h],
                recv_sem=ag_rsem.at[h],
                device_id=(right,),
                device_id_type=pl.DeviceIdType.MESH,
            )
            rdma.start()
            rdma.wait()

        def block_partial(xblk):
            q = jnp.dot(xblk, wq_ref[:, :], preferred_element_type=jnp.float32)
            outs = []
            for h in range(H):
                qh = q[:, h * DH:(h + 1) * DH]
                s = lax.dot_general(
                    qh, k_ref[h], (((1,), (1,)), ((), ())),
                    preferred_element_type=jnp.float32) * SCALE
                m = jnp.max(s, axis=1, keepdims=True)
                p = jnp.exp(s - m)
                l = jnp.sum(p, axis=1, keepdims=True)
                oh = jnp.dot(p, v_ref[h], preferred_element_type=jnp.float32)
                outs.append(oh / l)
            attn = jnp.concatenate(outs, axis=1)
            return jnp.dot(attn, wo_ref[:, :], preferred_element_type=jnp.float32)

        for s in range(N_DEV):
            partial[s, :, :] = block_partial(xfull[s])

        for t in range(N_DEV - 1):
            rdma = pltpu.make_async_remote_copy(
                src_ref=partial.at[t + 1],
                dst_ref=rsrecv.at[t],
                send_sem=rs_ssem.at[t],
                recv_sem=rs_rsem.at[t],
                device_id=(right,),
                device_id_type=pl.DeviceIdType.MESH,
            )
            rdma.start()
            rdma.wait()
            if t < N_DEV - 2:
                partial[t + 2, :, :] = partial[t + 2] + rsrecv[t]

        out_ref[:, :] = partial[0] + rsrecv[N_DEV - 2]

    out = pl.pallas_call(
        body,
        out_shape=jax.ShapeDtypeStruct((SQ, D), jnp.float32),
        in_specs=[pl.BlockSpec(memory_space=pltpu.VMEM)] * 5,
        out_specs=pl.BlockSpec(memory_space=pltpu.VMEM),
        scratch_shapes=[
            pltpu.VMEM((N_DEV, SQ, D), jnp.float32),
            pltpu.VMEM((N_DEV, SQ, D), jnp.float32),
            pltpu.VMEM((N_DEV - 1, SQ, D), jnp.float32),
            pltpu.SemaphoreType.DMA((N_DEV - 1,)),
            pltpu.SemaphoreType.DMA((N_DEV - 1,)),
            pltpu.SemaphoreType.DMA((N_DEV - 1,)),
            pltpu.SemaphoreType.DMA((N_DEV - 1,)),
        ],
        compiler_params=pltpu.CompilerParams(collective_id=0),
    )(xb, Wq, Wo, k, v)
    return out[None]


# baseline (device time: 285718 ns/iter reference)
import jax
import jax.numpy as jnp
from jax import lax
from jax.experimental import pallas as pl
from jax.experimental.pallas import tpu as pltpu

N_DEV = 4
SQ = 512
D = 1024
H = 8
DH = 128
SKV = 2048
SCALE = 0.08838834764831843


def kernel(x, Wq, Wo, K_ext, V_ext):
    my = lax.axis_index("i")
    xb = x[0]
    k = lax.dynamic_slice_in_dim(K_ext[0], my * H, H, axis=1)
    v = lax.dynamic_slice_in_dim(V_ext[0], my * H, H, axis=1)
    k = jnp.transpose(k, (1, 0, 2))
    v = jnp.transpose(v, (1, 0, 2))

    def body(x_ref, wq_ref, wo_ref, k_hbm, v_hbm, out_ref,
             xfull, partial, rsrecv, qbuf, attn, kbuf, vbuf,
             ag_ssem, ag_rsem, rs_ssem, rs_rsem, k_sems, v_sems):
        my_i = lax.axis_index("i")
        left = lax.rem(my_i - 1 + N_DEV, N_DEV)
        right = lax.rem(my_i + 1, N_DEV)

        barrier = pltpu.get_barrier_semaphore()
        for nbr in (left, right):
            pl.semaphore_signal(barrier, inc=1, device_id=(nbr,),
                                device_id_type=pl.DeviceIdType.MESH)
        pl.semaphore_wait(barrier, 2)

        for h in range(N_DEV - 1):
            rdma = pltpu.make_async_remote_copy(
                src_ref=x_ref if h == 0 else xfull.at[h - 1],
                dst_ref=xfull.at[h],
                send_sem=ag_ssem.at[h],
                recv_sem=ag_rsem.at[h],
                device_id=(right,),
                device_id_type=pl.DeviceIdType.MESH,
            )
            rdma.start()
            rdma.wait()

        def kv_copy(h, slot):
            return (
                pltpu.make_async_copy(k_hbm.at[h], kbuf.at[slot], k_sems.at[slot]),
                pltpu.make_async_copy(v_hbm.at[h], vbuf.at[slot], v_sems.at[slot]),
            )

        def block_partial(xsrc, dst_ref):
            for c in kv_copy(0, 0):
                c.start()
            qbuf[:, :] = jnp.dot(xsrc[:, :], wq_ref[:, :],
                                 preferred_element_type=jnp.float32)
            for h in range(H):
                cur = h % 2
                if h + 1 < H:
                    for c in kv_copy(h + 1, (h + 1) % 2):
                        c.start()
                for c in kv_copy(h, cur):
                    c.wait()
                qh = qbuf[:, h * DH:(h + 1) * DH]
                s = lax.dot_general(
                    qh, kbuf[cur], (((1,), (1,)), ((), ())),
                    preferred_element_type=jnp.float32) * SCALE
                m = jnp.max(s, axis=1, keepdims=True)
                p = jnp.exp(s - m)
                l = jnp.sum(p, axis=1, keepdims=True)
                oh = jnp.dot(p, vbuf[cur], preferred_element_type=jnp.float32)
                attn[:, h * DH:(h + 1) * DH] = oh / l
            dst_ref[:, :] = jnp.dot(attn[:, :], wo_ref[:, :],
                                    preferred_element_type=jnp.float32)

        block_partial(x_ref, out_ref)
        for j in range(N_DEV - 1):
            block_partial(xfull.at[j], partial.at[j])

        for t in range(N_DEV - 1):
            rdma = pltpu.make_async_remote_copy(
                src_ref=partial.at[t],
                dst_ref=rsrecv.at[t],
                send_sem=rs_ssem.at[t],
                recv_sem=rs_rsem.at[t],
                device_id=(right,),
                device_id_type=pl.DeviceIdType.MESH,
            )
            rdma.start()
            rdma.wait()
            if t < N_DEV - 2:
                partial[t + 1, :, :] = partial[t + 1] + rsrecv[t]

        out_ref[:, :] = out_ref[:, :] + rsrecv[N_DEV - 2]

    out = pl.pallas_call(
        body,
        out_shape=jax.ShapeDtypeStruct((SQ, D), jnp.float32),
        in_specs=[
            pl.BlockSpec(memory_space=pltpu.VMEM),
            pl.BlockSpec(memory_space=pltpu.VMEM),
            pl.BlockSpec(memory_space=pltpu.VMEM),
            pl.BlockSpec(memory_space=pltpu.MemorySpace.HBM),
            pl.BlockSpec(memory_space=pltpu.MemorySpace.HBM),
        ],
        out_specs=pl.BlockSpec(memory_space=pltpu.VMEM),
        scratch_shapes=[
            pltpu.VMEM((N_DEV - 1, SQ, D), jnp.float32),
            pltpu.VMEM((N_DEV - 1, SQ, D), jnp.float32),
            pltpu.VMEM((N_DEV - 1, SQ, D), jnp.float32),
            pltpu.VMEM((SQ, D), jnp.float32),
            pltpu.VMEM((SQ, D), jnp.float32),
            pltpu.VMEM((2, SKV, DH), jnp.float32),
            pltpu.VMEM((2, SKV, DH), jnp.float32),
            pltpu.SemaphoreType.DMA((N_DEV - 1,)),
            pltpu.SemaphoreType.DMA((N_DEV - 1,)),
            pltpu.SemaphoreType.DMA((N_DEV - 1,)),
            pltpu.SemaphoreType.DMA((N_DEV - 1,)),
            pltpu.SemaphoreType.DMA((2,)),
            pltpu.SemaphoreType.DMA((2,)),
        ],
        compiler_params=pltpu.CompilerParams(
            collective_id=0, vmem_limit_bytes=63 * 1024 * 1024),
    )(xb, Wq, Wo, k, v)
    return out[None]


# device time: 185867 ns/iter; 1.5372x vs baseline; 1.5372x over previous
import jax
import jax.numpy as jnp
from jax import lax
from jax.experimental import pallas as pl
from jax.experimental.pallas import tpu as pltpu

N_DEV = 4
SQ = 512
D = 1024
H = 8
DH = 128
SKV = 2048
SCALE = 0.08838834764831843


def kernel(x, Wq, Wo, K_ext, V_ext):
    my = lax.axis_index("i")
    xb = x[0]
    k = lax.dynamic_slice_in_dim(K_ext[0], my * H, H, axis=1)
    v = lax.dynamic_slice_in_dim(V_ext[0], my * H, H, axis=1)
    k = jnp.transpose(k, (1, 0, 2))
    v = jnp.transpose(v, (1, 0, 2))

    def body(x_ref, wq_ref, wo_ref, k_hbm, v_hbm, out_ref,
             xfull, partial, rsrecv, qbuf, attn, kbuf, vbuf,
             ag_ssem, ag_rsem, rs_ssem, rs_rsem, k_sems, v_sems):
        my_i = lax.axis_index("i")
        left = lax.rem(my_i - 1 + N_DEV, N_DEV)
        right = lax.rem(my_i + 1, N_DEV)

        barrier = pltpu.get_barrier_semaphore()
        for nbr in (left, right):
            pl.semaphore_signal(barrier, inc=1, device_id=(nbr,),
                                device_id_type=pl.DeviceIdType.MESH)
        pl.semaphore_wait(barrier, 2)

        def ag_rdma(h):
            return pltpu.make_async_remote_copy(
                src_ref=x_ref if h == 0 else xfull.at[h - 1],
                dst_ref=xfull.at[h],
                send_sem=ag_ssem.at[h],
                recv_sem=ag_rsem.at[h],
                device_id=(right,),
                device_id_type=pl.DeviceIdType.MESH,
            )

        def rs_rdma(t):
            return pltpu.make_async_remote_copy(
                src_ref=partial.at[t],
                dst_ref=rsrecv.at[t],
                send_sem=rs_ssem.at[t],
                recv_sem=rs_rsem.at[t],
                device_id=(right,),
                device_id_type=pl.DeviceIdType.MESH,
            )

        def kv_copy(h, slot):
            return (
                pltpu.make_async_copy(k_hbm.at[h], kbuf.at[slot], k_sems.at[slot]),
                pltpu.make_async_copy(v_hbm.at[h], vbuf.at[slot], v_sems.at[slot]),
            )

        def block_partial(xsrc, dst_ref):
            for c in kv_copy(0, 0):
                c.start()
            qbuf[:, :] = jnp.dot(xsrc[:, :], wq_ref[:, :],
                                 preferred_element_type=jnp.float32)
            for h in range(H):
                cur = h % 2
                if h + 1 < H:
                    for c in kv_copy(h + 1, (h + 1) % 2):
                        c.start()
                for c in kv_copy(h, cur):
                    c.wait()
                qh = qbuf[:, h * DH:(h + 1) * DH]
                s = lax.dot_general(
                    qh, kbuf[cur], (((1,), (1,)), ((), ())),
                    preferred_element_type=jnp.float32) * SCALE
                m = jnp.max(s, axis=1, keepdims=True)
                p = jnp.exp(s - m)
                l = jnp.sum(p, axis=1, keepdims=True)
                oh = jnp.dot(p, vbuf[cur], preferred_element_type=jnp.float32)
                attn[:, h * DH:(h + 1) * DH] = oh / l
            dst_ref[:, :] = jnp.dot(attn[:, :], wo_ref[:, :],
                                    preferred_element_type=jnp.float32)

        ag = [ag_rdma(h) for h in range(N_DEV - 1)]
        rs = [rs_rdma(t) for t in range(N_DEV - 1)]

        ag[0].start()
        block_partial(x_ref, out_ref)
        for j in range(N_DEV - 1):
            ag[j].wait_recv()
            if j + 1 < N_DEV - 1:
                ag[j + 1].start()
            block_partial(xfull.at[j], partial.at[j])
            if j > 0:
                rs[j - 1].wait_recv()
                partial[j, :, :] = partial[j] + rsrecv[j - 1]
            rs[j].start()
        rs[N_DEV - 2].wait_recv()
        out_ref[:, :] = out_ref[:, :] + rsrecv[N_DEV - 2]

        for r in ag + rs:
            r.wait_send()

    out = pl.pallas_call(
        body,
        out_shape=jax.ShapeDtypeStruct((SQ, D), jnp.float32),
        in_specs=[
            pl.BlockSpec(memory_space=pltpu.VMEM),
            pl.BlockSpec(memory_space=pltpu.VMEM),
            pl.BlockSpec(memory_space=pltpu.VMEM),
            pl.BlockSpec(memory_space=pltpu.MemorySpace.HBM),
            pl.BlockSpec(memory_space=pltpu.MemorySpace.HBM),
        ],
        out_specs=pl.BlockSpec(memory_space=pltpu.VMEM),
        scratch_shapes=[
            pltpu.VMEM((N_DEV - 1, SQ, D), jnp.float32),
            pltpu.VMEM((N_DEV - 1, SQ, D), jnp.float32),
            pltpu.VMEM((N_DEV - 1, SQ, D), jnp.float32),
            pltpu.VMEM((SQ, D), jnp.float32),
            pltpu.VMEM((SQ, D), jnp.float32),
            pltpu.VMEM((2, SKV, DH), jnp.float32),
            pltpu.VMEM((2, SKV, DH), jnp.float32),
            pltpu.SemaphoreType.DMA((N_DEV - 1,)),
            pltpu.SemaphoreType.DMA((N_DEV - 1,)),
            pltpu.SemaphoreType.DMA((N_DEV - 1,)),
            pltpu.SemaphoreType.DMA((N_DEV - 1,)),
            pltpu.SemaphoreType.DMA((2,)),
            pltpu.SemaphoreType.DMA((2,)),
        ],
        compiler_params=pltpu.CompilerParams(
            collective_id=0, vmem_limit_bytes=63 * 1024 * 1024),
    )(xb, Wq, Wo, k, v)
    return out[None]


# device time: 156775 ns/iter; 1.8225x vs baseline; 1.1856x over previous
import jax
import jax.numpy as jnp
from jax import lax
from jax.experimental import pallas as pl
from jax.experimental.pallas import tpu as pltpu

N_DEV = 4
SQ = 512
D = 1024
H = 8
DH = 128
SKV = 2048
SCALE = 0.08838834764831843
BF = jnp.bfloat16
F32 = jnp.float32


def kernel(x, Wq, Wo, K_ext, V_ext):
    my = lax.axis_index("i")
    xb = x[0].astype(BF)
    k = lax.dynamic_slice_in_dim(K_ext[0], my * H, H, axis=1)
    v = lax.dynamic_slice_in_dim(V_ext[0], my * H, H, axis=1)
    k = jnp.transpose(k, (1, 0, 2)).astype(BF)
    v = jnp.transpose(v, (1, 0, 2)).astype(BF)

    def body(x_ref, wq_ref, wo_ref, k_hbm, v_hbm, out_ref,
             xfull, partial, rsrecv, qbuf, attn, kbuf, vbuf,
             ag_ssem, ag_rsem, rs_ssem, rs_rsem, k_sems, v_sems):
        my_i = lax.axis_index("i")
        left = lax.rem(my_i - 1 + N_DEV, N_DEV)
        right = lax.rem(my_i + 1, N_DEV)

        barrier = pltpu.get_barrier_semaphore()
        for nbr in (left, right):
            pl.semaphore_signal(barrier, inc=1, device_id=(nbr,),
                                device_id_type=pl.DeviceIdType.MESH)
        pl.semaphore_wait(barrier, 2)

        def ag_rdma(h):
            return pltpu.make_async_remote_copy(
                src_ref=x_ref if h == 0 else xfull.at[h - 1],
                dst_ref=xfull.at[h],
                send_sem=ag_ssem.at[h],
                recv_sem=ag_rsem.at[h],
                device_id=(right,),
                device_id_type=pl.DeviceIdType.MESH,
            )

        def rs_rdma(t):
            return pltpu.make_async_remote_copy(
                src_ref=partial.at[t],
                dst_ref=rsrecv.at[t],
                send_sem=rs_ssem.at[t],
                recv_sem=rs_rsem.at[t],
                device_id=(right,),
                device_id_type=pl.DeviceIdType.MESH,
            )

        def kv_copy(h, slot):
            return (
                pltpu.make_async_copy(k_hbm.at[h], kbuf.at[slot], k_sems.at[slot]),
                pltpu.make_async_copy(v_hbm.at[h], vbuf.at[slot], v_sems.at[slot]),
            )

        def block_attn(xsrc, dst_ref, dst_f32):
            for c in kv_copy(0, 0):
                c.start()
            qbuf[:, :] = jnp.dot(xsrc[:, :], wq_ref[:, :],
                                 preferred_element_type=F32).astype(BF)
            for h in range(H):
                cur = h % 2
                if h + 1 < H:
                    for c in kv_copy(h + 1, (h + 1) % 2):
                        c.start()
                for c in kv_copy(h, cur):
                    c.wait()
                qh = qbuf[:, h * DH:(h + 1) * DH]
                s = lax.dot_general(
                    qh, kbuf[cur], (((1,), (1,)), ((), ())),
                    preferred_element_type=F32) * SCALE
                m = jnp.max(s, axis=1, keepdims=True)
                p = jnp.exp(s - m)
                l = jnp.sum(p, axis=1, keepdims=True)
                oh = jnp.dot(p.astype(BF), vbuf[cur],
                             preferred_element_type=F32)
                attn[:, h * DH:(h + 1) * DH] = (oh / l).astype(BF)
            o = jnp.dot(attn[:, :], wo_ref[:, :], preferred_element_type=F32)
            dst_ref[:, :] = o if dst_f32 else o.astype(BF)

        ag = [ag_rdma(h) for h in range(N_DEV - 1)]
        rs = [rs_rdma(t) for t in range(N_DEV - 1)]

        ag[0].start()
        block_attn(x_ref, out_ref, True)
        for j in range(N_DEV - 1):
            ag[j].wait_recv()
            if j + 1 < N_DEV - 1:
                ag[j + 1].start()
            block_attn(xfull.at[j], partial.at[j], False)
            if j > 0:
                rs[j - 1].wait_recv()
                partial[j, :, :] = (
                    partial[j].astype(F32) + rsrecv[j - 1].astype(F32)
                ).astype(BF)
            rs[j].start()
        rs[N_DEV - 2].wait_recv()
        out_ref[:, :] = out_ref[:, :] + rsrecv[N_DEV - 2].astype(F32)

        for r in ag + rs:
            r.wait_send()

    out = pl.pallas_call(
        body,
        out_shape=jax.ShapeDtypeStruct((SQ, D), F32),
        in_specs=[
            pl.BlockSpec(memory_space=pltpu.VMEM),
            pl.BlockSpec(memory_space=pltpu.VMEM),
            pl.BlockSpec(memory_space=pltpu.VMEM),
            pl.BlockSpec(memory_space=pltpu.MemorySpace.HBM),
            pl.BlockSpec(memory_space=pltpu.MemorySpace.HBM),
        ],
        out_specs=pl.BlockSpec(memory_space=pltpu.VMEM),
        scratch_shapes=[
            pltpu.VMEM((N_DEV - 1, SQ, D), BF),
            pltpu.VMEM((N_DEV - 1, SQ, D), BF),
            pltpu.VMEM((N_DEV - 1, SQ, D), BF),
            pltpu.VMEM((SQ, D), BF),
            pltpu.VMEM((SQ, D), BF),
            pltpu.VMEM((2, SKV, DH), BF),
            pltpu.VMEM((2, SKV, DH), BF),
            pltpu.SemaphoreType.DMA((N_DEV - 1,)),
            pltpu.SemaphoreType.DMA((N_DEV - 1,)),
            pltpu.SemaphoreType.DMA((N_DEV - 1,)),
            pltpu.SemaphoreType.DMA((N_DEV - 1,)),
            pltpu.SemaphoreType.DMA((2,)),
            pltpu.SemaphoreType.DMA((2,)),
        ],
        compiler_params=pltpu.CompilerParams(
            collective_id=0, vmem_limit_bytes=63 * 1024 * 1024),
    )(xb, Wq.astype(BF), Wo.astype(BF), k, v)
    return out[None]


# device time: 118084 ns/iter; 2.4196x vs baseline; 1.3277x over previous
import jax
import jax.numpy as jnp
from jax import lax
from jax.experimental import pallas as pl
from jax.experimental.pallas import tpu as pltpu

N_DEV = 4
SQ = 512
D = 1024
H = 8
DH = 128
SKV = 2048
SCALE = 0.08838834764831843
BF = jnp.bfloat16
F32 = jnp.float32


def kernel(x, Wq, Wo, K_ext, V_ext):
    my = lax.axis_index("i")
    xb = x[0].astype(BF)
    k = lax.dynamic_slice_in_dim(K_ext[0], my * H, H, axis=1)
    v = lax.dynamic_slice_in_dim(V_ext[0], my * H, H, axis=1)
    k = jnp.transpose(k, (1, 0, 2)).astype(BF)
    v = jnp.transpose(v, (1, 0, 2)).astype(BF)

    def body(x_ref, wq_ref, wo_ref, k_hbm, v_hbm, out_ref,
             xfull, partial, rsrecv, qbuf, attn, kbuf, vbuf,
             ag_ssem, ag_rsem, rs_ssem, rs_rsem, k_sems, v_sems):
        my_i = lax.axis_index("i")
        left = lax.rem(my_i - 1 + N_DEV, N_DEV)
        right = lax.rem(my_i + 1, N_DEV)

        barrier = pltpu.get_barrier_semaphore()
        for nbr in (left, right):
            pl.semaphore_signal(barrier, inc=1, device_id=(nbr,),
                                device_id_type=pl.DeviceIdType.MESH)
        pl.semaphore_wait(barrier, 2)

        def ag_rdma(h):
            return pltpu.make_async_remote_copy(
                src_ref=x_ref if h == 0 else xfull.at[h - 1],
                dst_ref=xfull.at[h],
                send_sem=ag_ssem.at[h],
                recv_sem=ag_rsem.at[h],
                device_id=(right,),
                device_id_type=pl.DeviceIdType.MESH,
            )

        def rs_rdma(t):
            return pltpu.make_async_remote_copy(
                src_ref=partial.at[t],
                dst_ref=rsrecv.at[t],
                send_sem=rs_ssem.at[t],
                recv_sem=rs_rsem.at[t],
                device_id=(right,),
                device_id_type=pl.DeviceIdType.MESH,
            )

        def kv_copy(h, slot):
            return (
                pltpu.make_async_copy(k_hbm.at[h], kbuf.at[slot], k_sems.at[slot]),
                pltpu.make_async_copy(v_hbm.at[h], vbuf.at[slot], v_sems.at[slot]),
            )

        def block_attn(xsrc, dst_ref, dst_f32):
            for c in kv_copy(0, 0):
                c.start()
            qbuf[:, :] = jnp.dot(xsrc[:, :], wq_ref[:, :],
                                 preferred_element_type=F32).astype(BF)
            for h in range(H):
                cur = h % 2
                if h + 1 < H:
                    for c in kv_copy(h + 1, (h + 1) % 2):
                        c.start()
                for c in kv_copy(h, cur):
                    c.wait()
                qh = qbuf[:, h * DH:(h + 1) * DH]
                s = lax.dot_general(
                    qh, kbuf[cur], (((1,), (1,)), ((), ())),
                    preferred_element_type=F32)
                p = jnp.exp(s).astype(BF)
                l = jnp.sum(p, axis=1, keepdims=True, dtype=F32)
                oh = jnp.dot(p, vbuf[cur], preferred_element_type=F32)
                attn[:, h * DH:(h + 1) * DH] = (oh / l).astype(BF)
            o = jnp.dot(attn[:, :], wo_ref[:, :], preferred_element_type=F32)
            dst_ref[:, :] = o if dst_f32 else o.astype(BF)

        ag = [ag_rdma(h) for h in range(N_DEV - 1)]
        rs = [rs_rdma(t) for t in range(N_DEV - 1)]

        ag[0].start()
        block_attn(x_ref, out_ref, True)
        for j in range(N_DEV - 1):
            ag[j].wait_recv()
            if j + 1 < N_DEV - 1:
                ag[j + 1].start()
            block_attn(xfull.at[j], partial.at[j], False)
            if j > 0:
                rs[j - 1].wait_recv()
                partial[j, :, :] = (
                    partial[j].astype(F32) + rsrecv[j - 1].astype(F32)
                ).astype(BF)
            rs[j].start()
        rs[N_DEV - 2].wait_recv()
        out_ref[:, :] = out_ref[:, :] + rsrecv[N_DEV - 2].astype(F32)

        for r in ag + rs:
            r.wait_send()

    out = pl.pallas_call(
        body,
        out_shape=jax.ShapeDtypeStruct((SQ, D), F32),
        in_specs=[
            pl.BlockSpec(memory_space=pltpu.VMEM),
            pl.BlockSpec(memory_space=pltpu.VMEM),
            pl.BlockSpec(memory_space=pltpu.VMEM),
            pl.BlockSpec(memory_space=pltpu.MemorySpace.HBM),
            pl.BlockSpec(memory_space=pltpu.MemorySpace.HBM),
        ],
        out_specs=pl.BlockSpec(memory_space=pltpu.VMEM),
        scratch_shapes=[
            pltpu.VMEM((N_DEV - 1, SQ, D), BF),
            pltpu.VMEM((N_DEV - 1, SQ, D), BF),
            pltpu.VMEM((N_DEV - 1, SQ, D), BF),
            pltpu.VMEM((SQ, D), BF),
            pltpu.VMEM((SQ, D), BF),
            pltpu.VMEM((2, SKV, DH), BF),
            pltpu.VMEM((2, SKV, DH), BF),
            pltpu.SemaphoreType.DMA((N_DEV - 1,)),
            pltpu.SemaphoreType.DMA((N_DEV - 1,)),
            pltpu.SemaphoreType.DMA((N_DEV - 1,)),
            pltpu.SemaphoreType.DMA((N_DEV - 1,)),
            pltpu.SemaphoreType.DMA((2,)),
            pltpu.SemaphoreType.DMA((2,)),
        ],
        compiler_params=pltpu.CompilerParams(
            collective_id=0, vmem_limit_bytes=63 * 1024 * 1024),
    )(xb, (Wq * SCALE).astype(BF), Wo.astype(BF), k, v)
    return out[None]


# device time: 97668 ns/iter; 2.9254x vs baseline; 1.2090x over previous
import jax
import jax.numpy as jnp
from jax import lax
from jax.experimental import pallas as pl
from jax.experimental.pallas import tpu as pltpu

N_DEV = 4
SQ = 512
D = 1024
H = 8
DH = 128
SKV = 2048
SCALE = 0.08838834764831843
BF = jnp.bfloat16
F32 = jnp.float32


def kernel(x, Wq, Wo, K_ext, V_ext):
    def body(x_ref, wq_ref, wo_ref, k_hbm, v_hbm, out_ref,
             xsend, xfull, partial, rsrecv, wqb, wob, qbuf, attn, kbuf, vbuf,
             ag_ssem, ag_rsem, rs_ssem, rs_rsem, k_sems, v_sems):
        my_i = lax.axis_index("i")
        left = lax.rem(my_i - 1 + N_DEV, N_DEV)
        right = lax.rem(my_i + 1, N_DEV)
        h0 = my_i * H

        barrier = pltpu.get_barrier_semaphore()
        for nbr in (left, right):
            pl.semaphore_signal(barrier, inc=1, device_id=(nbr,),
                                device_id_type=pl.DeviceIdType.MESH)
        xsend[:, :] = x_ref[:, :].astype(BF)
        pl.semaphore_wait(barrier, 2)

        def ag_rdma(h):
            return pltpu.make_async_remote_copy(
                src_ref=xsend if h == 0 else xfull.at[h - 1],
                dst_ref=xfull.at[h],
                send_sem=ag_ssem.at[h],
                recv_sem=ag_rsem.at[h],
                device_id=(right,),
                device_id_type=pl.DeviceIdType.MESH,
            )

        def rs_rdma(t):
            return pltpu.make_async_remote_copy(
                src_ref=partial.at[t],
                dst_ref=rsrecv.at[t],
                send_sem=rs_ssem.at[t],
                recv_sem=rs_rsem.at[t],
                device_id=(right,),
                device_id_type=pl.DeviceIdType.MESH,
            )

        def kv_copy(h, slot):
            return (
                pltpu.make_async_copy(
                    k_hbm.at[:, h0 + h, :], kbuf.at[slot], k_sems.at[slot]),
                pltpu.make_async_copy(
                    v_hbm.at[:, h0 + h, :], vbuf.at[slot], v_sems.at[slot]),
            )

        def block_attn(xsrc, dst_ref, dst_f32):
            for c in kv_copy(0, 0):
                c.start()
            qbuf[:, :] = jnp.dot(xsrc[:, :], wqb[:, :],
                                 preferred_element_type=F32).astype(BF)
            for h in range(H):
                cur = h % 2
                if h + 1 < H:
                    for c in kv_copy(h + 1, (h + 1) % 2):
                        c.start()
                for c in kv_copy(h, cur):
                    c.wait()
                qh = qbuf[:, h * DH:(h + 1) * DH]
                s = lax.dot_general(
                    qh, kbuf[cur].astype(BF), (((1,), (1,)), ((), ())),
                    preferred_element_type=F32)
                p = jnp.exp(s).astype(BF)
                l = jnp.sum(p, axis=1, keepdims=True, dtype=F32)
                oh = jnp.dot(p, vbuf[cur].astype(BF),
                             preferred_element_type=F32)
                attn[:, h * DH:(h + 1) * DH] = (oh / l).astype(BF)
            o = jnp.dot(attn[:, :], wob[:, :], preferred_element_type=F32)
            dst_ref[:, :] = o if dst_f32 else o.astype(BF)

        ag = [ag_rdma(h) for h in range(N_DEV - 1)]
        rs = [rs_rdma(t) for t in range(N_DEV - 1)]

        ag[0].start()
        wqb[:, :] = (wq_ref[:, :] * SCALE).astype(BF)
        wob[:, :] = wo_ref[:, :].astype(BF)
        block_attn(xsend, out_ref, True)
        for j in range(N_DEV - 1):
            ag[j].wait_recv()
            if j + 1 < N_DEV - 1:
                ag[j + 1].start()
            block_attn(xfull.at[j], partial.at[j], False)
            if j > 0:
                rs[j - 1].wait_recv()
                partial[j, :, :] = (
                    partial[j].astype(F32) + rsrecv[j - 1].astype(F32)
                ).astype(BF)
            rs[j].start()
        rs[N_DEV - 2].wait_recv()
        out_ref[:, :] = out_ref[:, :] + rsrecv[N_DEV - 2].astype(F32)

        for r in ag + rs:
            r.wait_send()

    out = pl.pallas_call(
        body,
        out_shape=jax.ShapeDtypeStruct((SQ, D), F32),
        in_specs=[
            pl.BlockSpec(memory_space=pltpu.VMEM),
            pl.BlockSpec(memory_space=pltpu.VMEM),
            pl.BlockSpec(memory_space=pltpu.VMEM),
            pl.BlockSpec(memory_space=pltpu.MemorySpace.HBM),
            pl.BlockSpec(memory_space=pltpu.MemorySpace.HBM),
        ],
        out_specs=pl.BlockSpec(memory_space=pltpu.VMEM),
        scratch_shapes=[
            pltpu.VMEM((SQ, D), BF),
            pltpu.VMEM((N_DEV - 1, SQ, D), BF),
            pltpu.VMEM((N_DEV - 1, SQ, D), BF),
            pltpu.VMEM((N_DEV - 1, SQ, D), BF),
            pltpu.VMEM((D, D), BF),
            pltpu.VMEM((D, D), BF),
            pltpu.VMEM((SQ, D), BF),
            pltpu.VMEM((SQ, D), BF),
            pltpu.VMEM((2, SKV, DH), F32),
            pltpu.VMEM((2, SKV, DH), F32),
            pltpu.SemaphoreType.DMA((N_DEV - 1,)),
            pltpu.SemaphoreType.DMA((N_DEV - 1,)),
            pltpu.SemaphoreType.DMA((N_DEV - 1,)),
            pltpu.SemaphoreType.DMA((N_DEV - 1,)),
            pltpu.SemaphoreType.DMA((2,)),
            pltpu.SemaphoreType.DMA((2,)),
        ],
        compiler_params=pltpu.CompilerParams(
            collective_id=0, vmem_limit_bytes=63 * 1024 * 1024),
    )(x[0], Wq, Wo, K_ext[0], V_ext[0])
    return out[None]


# device time: 96549 ns/iter; 2.9593x vs baseline; 1.0116x over previous
import jax
import jax.numpy as jnp
from jax import lax
from jax.experimental import pallas as pl
from jax.experimental.pallas import tpu as pltpu

N_DEV = 4
SQ = 512
D = 1024
H = 8
DH = 128
SKV = 2048
SCALE = 0.08838834764831843
NCH = 4
CH = SQ // NCH
BF = jnp.bfloat16
F32 = jnp.float32


def kernel(x, Wq, Wo, K_ext, V_ext):
    def body(x_ref, wq_ref, wo_ref, k_hbm, v_hbm, out_ref,
             xsend, xfull, partial, rsrecv, wqb, wob, qbuf, attn, kbuf, vbuf,
             ag_ssem, ag_rsem, rs_ssem, rs_rsem, rs2_ssem, rs2_rsem,
             k_sems, v_sems):
        my_i = lax.axis_index("i")
        left = lax.rem(my_i - 1 + N_DEV, N_DEV)
        right = lax.rem(my_i + 1, N_DEV)
        h0 = my_i * H

        barrier = pltpu.get_barrier_semaphore()
        for nbr in (left, right):
            pl.semaphore_signal(barrier, inc=1, device_id=(nbr,),
                                device_id_type=pl.DeviceIdType.MESH)
        xsend[:, :] = x_ref[:, :].astype(BF)
        pl.semaphore_wait(barrier, 2)

        def ag_rdma(h):
            return pltpu.make_async_remote_copy(
                src_ref=xsend if h == 0 else xfull.at[h - 1],
                dst_ref=xfull.at[h],
                send_sem=ag_ssem.at[h],
                recv_sem=ag_rsem.at[h],
                device_id=(right,),
                device_id_type=pl.DeviceIdType.MESH,
            )

        def rs_rdma(t):
            return pltpu.make_async_remote_copy(
                src_ref=partial.at[t],
                dst_ref=rsrecv.at[t],
                send_sem=rs_ssem.at[t],
                recv_sem=rs_rsem.at[t],
                device_id=(right,),
                device_id_type=pl.DeviceIdType.MESH,
            )

        def kv_copy(h, slot):
            return (
                pltpu.make_async_copy(
                    k_hbm.at[:, h0 + h, :], kbuf.at[slot], k_sems.at[slot]),
                pltpu.make_async_copy(
                    v_hbm.at[:, h0 + h, :], vbuf.at[slot], v_sems.at[slot]),
            )

        def attn_core(xsrc):
            for c in kv_copy(0, 0):
                c.start()
            qbuf[:, :] = jnp.dot(xsrc[:, :], wqb[:, :],
                                 preferred_element_type=F32).astype(BF)
            for h in range(H):
                cur = h % 2
                if h + 1 < H:
                    for c in kv_copy(h + 1, (h + 1) % 2):
                        c.start()
                for c in kv_copy(h, cur):
                    c.wait()
                qh = qbuf[:, h * DH:(h + 1) * DH]
                s = lax.dot_general(
                    qh, kbuf[cur].astype(BF), (((1,), (1,)), ((), ())),
                    preferred_element_type=F32)
                p = jnp.exp(s).astype(BF)
                l = jnp.sum(p, axis=1, keepdims=True, dtype=F32)
                oh = jnp.dot(p, vbuf[cur].astype(BF),
                             preferred_element_type=F32)
                attn[:, h * DH:(h + 1) * DH] = (oh / l).astype(BF)

        def block_attn(xsrc, dst_ref, dst_f32):
            attn_core(xsrc)
            o = jnp.dot(attn[:, :], wob[:, :], preferred_element_type=F32)
            dst_ref[:, :] = o if dst_f32 else o.astype(BF)

        ag = [ag_rdma(h) for h in range(N_DEV - 1)]
        rs = [rs_rdma(t) for t in range(N_DEV - 1)]

        ag[0].start()
        wqb[:, :] = (wq_ref[:, :] * SCALE).astype(BF)
        wob[:, :] = wo_ref[:, :].astype(BF)
        block_attn(xsend, out_ref, True)
        for j in range(N_DEV - 2):
            ag[j].wait_recv()
            ag[j + 1].start()
            block_attn(xfull.at[j], partial.at[j], False)
            if j > 0:
                rs[j - 1].wait_recv()
                partial[j, :, :] = (
                    partial[j].astype(F32) + rsrecv[j - 1].astype(F32)
                ).astype(BF)
            rs[j].start()

        last = N_DEV - 2
        ag[last].wait_recv()
        attn_core(xfull.at[last])
        rs[last - 1].wait_recv()
        rs2 = [
            pltpu.make_async_remote_copy(
                src_ref=partial.at[last, pl.ds(c * CH, CH)],
                dst_ref=rsrecv.at[last, pl.ds(c * CH, CH)],
                send_sem=rs2_ssem.at[c],
                recv_sem=rs2_rsem.at[c],
                device_id=(right,),
                device_id_type=pl.DeviceIdType.MESH,
            )
            for c in range(NCH)
        ]
        for c in range(NCH):
            r0, r1 = c * CH, (c + 1) * CH
            o = jnp.dot(attn[r0:r1, :], wob[:, :], preferred_element_type=F32)
            partial[last, r0:r1, :] = (
                o + rsrecv[last - 1, r0:r1, :].astype(F32)
            ).astype(BF)
            rs2[c].start()
        for c in range(NCH):
            r0, r1 = c * CH, (c + 1) * CH
            rs2[c].wait_recv()
            out_ref[r0:r1, :] = (
                out_ref[r0:r1, :] + rsrecv[last, r0:r1, :].astype(F32)
            )

        for r in ag + rs[:last] + rs2:
            r.wait_send()

    out = pl.pallas_call(
        body,
        out_shape=jax.ShapeDtypeStruct((SQ, D), F32),
        in_specs=[
            pl.BlockSpec(memory_space=pltpu.VMEM),
            pl.BlockSpec(memory_space=pltpu.VMEM),
            pl.BlockSpec(memory_space=pltpu.VMEM),
            pl.BlockSpec(memory_space=pltpu.MemorySpace.HBM),
            pl.BlockSpec(memory_space=pltpu.MemorySpace.HBM),
        ],
        out_specs=pl.BlockSpec(memory_space=pltpu.VMEM),
        scratch_shapes=[
            pltpu.VMEM((SQ, D), BF),
            pltpu.VMEM((N_DEV - 1, SQ, D), BF),
            pltpu.VMEM((N_DEV - 1, SQ, D), BF),
            pltpu.VMEM((N_DEV - 1, SQ, D), BF),
            pltpu.VMEM((D, D), BF),
            pltpu.VMEM((D, D), BF),
            pltpu.VMEM((SQ, D), BF),
            pltpu.VMEM((SQ, D), BF),
            pltpu.VMEM((2, SKV, DH), F32),
            pltpu.VMEM((2, SKV, DH), F32),
            pltpu.SemaphoreType.DMA((N_DEV - 1,)),
            pltpu.SemaphoreType.DMA((N_DEV - 1,)),
            pltpu.SemaphoreType.DMA((N_DEV - 1,)),
            pltpu.SemaphoreType.DMA((N_DEV - 1,)),
            pltpu.SemaphoreType.DMA((NCH,)),
            pltpu.SemaphoreType.DMA((NCH,)),
            pltpu.SemaphoreType.DMA((2,)),
            pltpu.SemaphoreType.DMA((2,)),
        ],
        compiler_params=pltpu.CompilerParams(
            collective_id=0, vmem_limit_bytes=63 * 1024 * 1024),
    )(x[0], Wq, Wo, K_ext[0], V_ext[0])
    return out[None]


# device time: 91678 ns/iter; 3.1165x vs baseline; 1.0531x over previous
import jax
import jax.numpy as jnp
from jax import lax
from jax.experimental import pallas as pl
from jax.experimental.pallas import tpu as pltpu

N_DEV = 4
SQ = 512
D = 1024
H = 8
DH = 128
SKV = 2048
SCALE = 0.08838834764831843
NCH = 4
CH = SQ // NCH
BF = jnp.bfloat16
F32 = jnp.float32


def kernel(x, Wq, Wo, K_ext, V_ext):
    def body(x_ref, wq_ref, wo_ref, k_hbm, v_hbm, out_ref,
             xsend, xfull, partial, rsrecv, wqb, wob, qbuf, attn,
             kbuf, vbuf, kall, vall,
             ag_ssem, ag_rsem, rs_ssem, rs_rsem, rs2_ssem, rs2_rsem,
             k_sems, v_sems):
        my_i = lax.axis_index("i")
        left = lax.rem(my_i - 1 + N_DEV, N_DEV)
        right = lax.rem(my_i + 1, N_DEV)
        h0 = my_i * H

        barrier = pltpu.get_barrier_semaphore()
        for nbr in (left, right):
            pl.semaphore_signal(barrier, inc=1, device_id=(nbr,),
                                device_id_type=pl.DeviceIdType.MESH)
        xsend[:, :] = x_ref[:, :].astype(BF)
        pl.semaphore_wait(barrier, 2)

        def ag_rdma(h):
            return pltpu.make_async_remote_copy(
                src_ref=xsend if h == 0 else xfull.at[h - 1],
                dst_ref=xfull.at[h],
                send_sem=ag_ssem.at[h],
                recv_sem=ag_rsem.at[h],
                device_id=(right,),
                device_id_type=pl.DeviceIdType.MESH,
            )

        def rs_rdma(t):
            return pltpu.make_async_remote_copy(
                src_ref=partial.at[t],
                dst_ref=rsrecv.at[t],
                send_sem=rs_ssem.at[t],
                recv_sem=rs_rsem.at[t],
                device_id=(right,),
                device_id_type=pl.DeviceIdType.MESH,
            )

        def kv_copy(h, slot):
            return (
                pltpu.make_async_copy(
                    k_hbm.at[:, h0 + h, :], kbuf.at[slot], k_sems.at[slot]),
                pltpu.make_async_copy(
                    v_hbm.at[:, h0 + h, :], vbuf.at[slot], v_sems.at[slot]),
            )

        def attn_core(xsrc, stream):
            if stream:
                for c in kv_copy(0, 0):
                    c.start()
            qbuf[:, :] = jnp.dot(xsrc[:, :], wqb[:, :],
                                 preferred_element_type=F32).astype(BF)
            for h in range(H):
                if stream:
                    cur = h % 2
                    if h + 1 < H:
                        for c in kv_copy(h + 1, (h + 1) % 2):
                            c.start()
                    for c in kv_copy(h, cur):
                        c.wait()
                    kall[h, :, :] = kbuf[cur].astype(BF)
                    vall[h, :, :] = vbuf[cur].astype(BF)
                qh = qbuf[:, h * DH:(h + 1) * DH]
                s = lax.dot_general(
                    qh, kall[h], (((1,), (1,)), ((), ())),
                    preferred_element_type=F32)
                p = jnp.exp(s).astype(BF)
                l = jnp.sum(p, axis=1, keepdims=True, dtype=F32)
                oh = jnp.dot(p, vall[h], preferred_element_type=F32)
                attn[:, h * DH:(h + 1) * DH] = (oh / l).astype(BF)

        def block_attn(xsrc, dst_ref, dst_f32, stream=False):
            attn_core(xsrc, stream)
            o = jnp.dot(attn[:, :], wob[:, :], preferred_element_type=F32)
            dst_ref[:, :] = o if dst_f32 else o.astype(BF)

        ag = [ag_rdma(h) for h in range(N_DEV - 1)]
        rs = [rs_rdma(t) for t in range(N_DEV - 1)]

        ag[0].start()
        wqb[:, :] = (wq_ref[:, :] * SCALE).astype(BF)
        wob[:, :] = wo_ref[:, :].astype(BF)
        block_attn(xsend, out_ref, True, stream=True)
        for j in range(N_DEV - 2):
            ag[j].wait_recv()
            ag[j + 1].start()
            block_attn(xfull.at[j], partial.at[j], False)
            if j > 0:
                rs[j - 1].wait_recv()
                partial[j, :, :] = (
                    partial[j].astype(F32) + rsrecv[j - 1].astype(F32)
                ).astype(BF)
            rs[j].start()

        last = N_DEV - 2
        ag[last].wait_recv()
        attn_core(xfull.at[last], False)
        rs[last - 1].wait_recv()
        rs2 = [
            pltpu.make_async_remote_copy(
                src_ref=partial.at[last, pl.ds(c * CH, CH)],
                dst_ref=rsrecv.at[last, pl.ds(c * CH, CH)],
                send_sem=rs2_ssem.at[c],
                recv_sem=rs2_rsem.at[c],
                device_id=(right,),
                device_id_type=pl.DeviceIdType.MESH,
            )
            for c in range(NCH)
        ]
        for c in range(NCH):
            r0, r1 = c * CH, (c + 1) * CH
            o = jnp.dot(attn[r0:r1, :], wob[:, :], preferred_element_type=F32)
            partial[last, r0:r1, :] = (
                o + rsrecv[last - 1, r0:r1, :].astype(F32)
            ).astype(BF)
            rs2[c].start()
        for c in range(NCH):
            r0, r1 = c * CH, (c + 1) * CH
            rs2[c].wait_recv()
            out_ref[r0:r1, :] = (
                out_ref[r0:r1, :] + rsrecv[last, r0:r1, :].astype(F32)
            )

        for r in ag + rs[:last] + rs2:
            r.wait_send()

    out = pl.pallas_call(
        body,
        out_shape=jax.ShapeDtypeStruct((SQ, D), F32),
        in_specs=[
            pl.BlockSpec(memory_space=pltpu.VMEM),
            pl.BlockSpec(memory_space=pltpu.VMEM),
            pl.BlockSpec(memory_space=pltpu.VMEM),
            pl.BlockSpec(memory_space=pltpu.MemorySpace.HBM),
            pl.BlockSpec(memory_space=pltpu.MemorySpace.HBM),
        ],
        out_specs=pl.BlockSpec(memory_space=pltpu.VMEM),
        scratch_shapes=[
            pltpu.VMEM((SQ, D), BF),
            pltpu.VMEM((N_DEV - 1, SQ, D), BF),
            pltpu.VMEM((N_DEV - 1, SQ, D), BF),
            pltpu.VMEM((N_DEV - 1, SQ, D), BF),
            pltpu.VMEM((D, D), BF),
            pltpu.VMEM((D, D), BF),
            pltpu.VMEM((SQ, D), BF),
            pltpu.VMEM((SQ, D), BF),
            pltpu.VMEM((2, SKV, DH), F32),
            pltpu.VMEM((2, SKV, DH), F32),
            pltpu.VMEM((H, SKV, DH), BF),
            pltpu.VMEM((H, SKV, DH), BF),
            pltpu.SemaphoreType.DMA((N_DEV - 1,)),
            pltpu.SemaphoreType.DMA((N_DEV - 1,)),
            pltpu.SemaphoreType.DMA((N_DEV - 1,)),
            pltpu.SemaphoreType.DMA((N_DEV - 1,)),
            pltpu.SemaphoreType.DMA((NCH,)),
            pltpu.SemaphoreType.DMA((NCH,)),
            pltpu.SemaphoreType.DMA((2,)),
            pltpu.SemaphoreType.DMA((2,)),
        ],
        compiler_params=pltpu.CompilerParams(
            collective_id=0, vmem_limit_bytes=63 * 1024 * 1024),
    )(x[0], Wq, Wo, K_ext[0], V_ext[0])
    return out[None]
